# baseline (speedup 1.0000x reference)
import functools

import jax
import jax.numpy as jnp
from jax import lax
from jax.experimental import pallas as pl
from jax.experimental.pallas import tpu as pltpu
from jax.experimental.pallas import tpu_sc as plsc

_NUM_GROUP = 512
_GROUP_SIZE = 32


def _fps_body(xt_ref, idx_ref, cen_ref, dists_ref, *, n_samples):
    B, N = xt_ref.shape[1], xt_ref.shape[2]
    xs = xt_ref[0]
    ys = xt_ref[1]
    zs = xt_ref[2]
    liota = lax.broadcasted_iota(jnp.int32, (B, N), 1)
    giota = lax.broadcasted_iota(jnp.int32, (B, n_samples), 1)
    dists_ref[...] = jnp.full((B, N), 1e10, dtype=jnp.float32)

    def body(i, carry):
        dists = dists_ref[...]
        m = jnp.max(dists, axis=1, keepdims=True)
        far = jnp.min(jnp.where(dists == m, liota, jnp.int32(N)),
                      axis=1, keepdims=True)
        sel = giota == i
        idx_ref[...] = jnp.where(sel, far, idx_ref[...])
        onehot = liota == far
        zero = jnp.zeros((B, N), jnp.float32)
        cx = jnp.sum(jnp.where(onehot, xs, zero), axis=1, keepdims=True)
        cy = jnp.sum(jnp.where(onehot, ys, zero), axis=1, keepdims=True)
        cz = jnp.sum(jnp.where(onehot, zs, zero), axis=1, keepdims=True)
        cen_ref[0] = jnp.where(sel, cx, cen_ref[0])
        cen_ref[1] = jnp.where(sel, cy, cen_ref[1])
        cen_ref[2] = jnp.where(sel, cz, cen_ref[2])
        dx = xs - cx
        dy = ys - cy
        dz = zs - cz
        d = (dx * dx + dy * dy) + dz * dz
        dists_ref[...] = jnp.minimum(dists, d)
        return carry

    lax.fori_loop(0, n_samples, body, 0)


def _fps_pallas(xt, n_samples):
    _, B, N = xt.shape
    body = functools.partial(_fps_body, n_samples=n_samples)
    idx, cen = pl.pallas_call(
        body,
        out_shape=(
            jax.ShapeDtypeStruct((B, n_samples), jnp.int32),
            jax.ShapeDtypeStruct((3, B, n_samples), jnp.float32),
        ),
        scratch_shapes=[pltpu.VMEM((B, N), jnp.float32)],
    )(xt)
    return idx, cen


def _knn_body(cen_ref, xt_ref, c2_ref, x2_ref, idx_ref, *, k):
    Gt = cen_ref.shape[2]
    N = xt_ref.shape[2]
    c = cen_ref[0]
    xt = xt_ref[0]
    cx = lax.dot_general(c, xt, (((0,), (0,)), ((), ())),
                         preferred_element_type=jnp.float32)
    d = (c2_ref[0] - 2.0 * cx) + x2_ref[0]
    iota = lax.broadcasted_iota(jnp.int32, (Gt, N), 1)
    cols = []
    for _ in range(k):
        m = jnp.min(d, axis=1, keepdims=True)
        amin = jnp.min(jnp.where(d == m, iota, jnp.int32(N)),
                       axis=1, keepdims=True)
        cols.append(amin)
        d = jnp.where(iota == amin, jnp.float32(jnp.inf), d)
    idx_ref[0] = jnp.concatenate(cols, axis=1)


def _knn_pallas(cen, xt, c2, x2, k, g_tile):
    B, _, G = cen.shape
    N = xt.shape[2]
    body = functools.partial(_knn_body, k=k)
    return pl.pallas_call(
        body,
        grid=(B, G // g_tile),
        in_specs=[
            pl.BlockSpec((1, 3, g_tile), lambda b, g: (b, 0, g)),
            pl.BlockSpec((1, 3, N), lambda b, g: (b, 0, 0)),
            pl.BlockSpec((1, g_tile, 1), lambda b, g: (b, g, 0)),
            pl.BlockSpec((1, 1, N), lambda b, g: (b, 0, 0)),
        ],
        out_specs=pl.BlockSpec((1, g_tile, k), lambda b, g: (b, g, 0)),
        out_shape=jax.ShapeDtypeStruct((B, G, k), jnp.int32),
        compiler_params=pltpu.CompilerParams(
            dimension_semantics=("parallel", "parallel")),
    )(cen, xt, c2, x2)


_SC_CHUNK = 128


def _sc_gather(table, gidx2d):
    nrows = gidx2d.shape[0]
    info = plsc.get_sparse_core_info()
    nw = info.num_cores * info.num_subcores
    nch = nrows // nw

    mesh = plsc.VectorSubcoreMesh(core_axis_name="c", subcore_axis_name="s")

    @functools.partial(
        pl.kernel,
        out_type=jax.ShapeDtypeStruct((nrows, _SC_CHUNK, 16), jnp.float32),
        mesh=mesh,
        scratch_types=[
            pltpu.VMEM((nch, _SC_CHUNK), jnp.int32),
            pltpu.VMEM((nch, _SC_CHUNK, 16), jnp.float32),
            pltpu.SemaphoreType.DMA,
        ],
        compiler_params=pltpu.CompilerParams(use_tc_tiling_on_sc=False),
    )
    def gat(table_hbm, idx_hbm, out_hbm, idx_v, rows_v, sem):
        wid = lax.axis_index("s") * info.num_cores + lax.axis_index("c")
        base = wid * nch
        pltpu.sync_copy(idx_hbm.at[pl.ds(base, nch)], idx_v)
        handles = [
            pltpu.async_copy(table_hbm.at[idx_v.at[j]], rows_v.at[j], sem)
            for j in range(nch)
        ]
        for h in handles:
            h.wait()
        pltpu.sync_copy(rows_v, out_hbm.at[pl.ds(base, nch)])

    return gat(table, gidx2d)


def kernel(xyz):
    B, N, C = xyz.shape
    xyz_only = xyz[:, :, :3]
    xt = jnp.transpose(xyz_only, (2, 0, 1))
    fps_idx, cen = _fps_pallas(xt, _NUM_GROUP)
    cen_b = jnp.transpose(cen, (1, 0, 2))
    xt_b = jnp.transpose(xyz_only, (0, 2, 1))
    center = jnp.transpose(cen, (1, 2, 0))
    c2 = jnp.sum(center ** 2, axis=-1)[:, :, None]
    x2 = jnp.sum(xyz_only ** 2, axis=-1)[:, None, :]
    idx = _knn_pallas(cen_b, xt_b, c2, x2, _GROUP_SIZE, 256)
    table = jnp.pad(xyz_only.reshape(B * N, 3), ((0, 0), (0, 13)))
    b2 = jnp.arange(B, dtype=jnp.int32)[:, None, None]
    gidx = (b2 * N + idx).reshape(-1, _SC_CHUNK)
    rows = _sc_gather(table, gidx)
    neigh_xyz = rows[..., :3].reshape(B, _NUM_GROUP, _GROUP_SIZE, 3)
    neighborhood = neigh_xyz - center[:, :, None, :]
    return (neighborhood, center, idx)


# speedup vs baseline: 1.0714x; 1.0714x over previous
import functools

import jax
import jax.numpy as jnp
from jax import lax
from jax.experimental import pallas as pl
from jax.experimental.pallas import tpu as pltpu
from jax.experimental.pallas import tpu_sc as plsc

_NUM_GROUP = 512
_GROUP_SIZE = 32


def _fps_body(xt_ref, idx_ref, cen_ref, dists_ref, *, n_samples):
    B, N = xt_ref.shape[1], xt_ref.shape[2]
    xs = xt_ref[0]
    ys = xt_ref[1]
    zs = xt_ref[2]
    liota = lax.broadcasted_iota(jnp.int32, (B, N), 1)
    giota = lax.broadcasted_iota(jnp.int32, (B, n_samples), 1)
    dists_ref[...] = jnp.full((B, N), 1e10, dtype=jnp.float32)

    def body(i, carry):
        dists = dists_ref[...]
        m = jnp.max(dists, axis=1, keepdims=True)
        far = jnp.min(jnp.where(dists == m, liota, jnp.int32(N)),
                      axis=1, keepdims=True)
        sel = giota == i
        idx_ref[...] = jnp.where(sel, far, idx_ref[...])
        onehot = liota == far
        zero = jnp.zeros((B, N), jnp.float32)
        cx = jnp.sum(jnp.where(onehot, xs, zero), axis=1, keepdims=True)
        cy = jnp.sum(jnp.where(onehot, ys, zero), axis=1, keepdims=True)
        cz = jnp.sum(jnp.where(onehot, zs, zero), axis=1, keepdims=True)
        cen_ref[0] = jnp.where(sel, cx, cen_ref[0])
        cen_ref[1] = jnp.where(sel, cy, cen_ref[1])
        cen_ref[2] = jnp.where(sel, cz, cen_ref[2])
        dx = xs - cx
        dy = ys - cy
        dz = zs - cz
        d = (dx * dx + dy * dy) + dz * dz
        dists_ref[...] = jnp.minimum(dists, d)
        return carry

    lax.fori_loop(0, n_samples, body, 0)


def _fps_pallas(xt, n_samples):
    _, B, N = xt.shape
    body = functools.partial(_fps_body, n_samples=n_samples)
    idx, cen = pl.pallas_call(
        body,
        out_shape=(
            jax.ShapeDtypeStruct((B, n_samples), jnp.int32),
            jax.ShapeDtypeStruct((3, B, n_samples), jnp.float32),
        ),
        scratch_shapes=[pltpu.VMEM((B, N), jnp.float32)],
    )(xt)
    return idx, cen


def _knn_body(cen_ref, xt_ref, c2_ref, x2_ref, idx_ref, *, k):
    Gt = cen_ref.shape[2]
    N = xt_ref.shape[2]
    c = cen_ref[0]
    xt = xt_ref[0]
    cx = lax.dot_general(c, xt, (((0,), (0,)), ((), ())),
                         preferred_element_type=jnp.float32)
    d = (c2_ref[0] - 2.0 * cx) + x2_ref[0]
    iota = lax.broadcasted_iota(jnp.int32, (Gt, N), 1)
    cols = []
    for _ in range(k):
        m = jnp.min(d, axis=1, keepdims=True)
        amin = jnp.min(jnp.where(d == m, iota, jnp.int32(N)),
                       axis=1, keepdims=True)
        cols.append(amin)
        d = jnp.where(iota == amin, jnp.float32(jnp.inf), d)
    idx_ref[0] = jnp.concatenate(cols, axis=1)


def _knn_pallas(cen, xt, c2, x2, k, g_tile):
    B, _, G = cen.shape
    N = xt.shape[2]
    body = functools.partial(_knn_body, k=k)
    return pl.pallas_call(
        body,
        grid=(B, G // g_tile),
        in_specs=[
            pl.BlockSpec((1, 3, g_tile), lambda b, g: (b, 0, g)),
            pl.BlockSpec((1, 3, N), lambda b, g: (b, 0, 0)),
            pl.BlockSpec((1, g_tile, 1), lambda b, g: (b, g, 0)),
            pl.BlockSpec((1, 1, N), lambda b, g: (b, 0, 0)),
        ],
        out_specs=pl.BlockSpec((1, g_tile, k), lambda b, g: (b, g, 0)),
        out_shape=jax.ShapeDtypeStruct((B, G, k), jnp.int32),
        compiler_params=pltpu.CompilerParams(
            dimension_semantics=("parallel", "parallel")),
    )(cen, xt, c2, x2)


_SC_CHUNK = 128


def _sc_gather(table, gidx2d):
    nrows = gidx2d.shape[0]
    info = plsc.get_sparse_core_info()
    nw = info.num_cores * info.num_subcores
    nch = nrows // nw

    mesh = plsc.VectorSubcoreMesh(core_axis_name="c", subcore_axis_name="s")

    @functools.partial(
        pl.kernel,
        out_type=jax.ShapeDtypeStruct((nrows, _SC_CHUNK, 16), jnp.float32),
        mesh=mesh,
        scratch_types=[
            pltpu.VMEM((nch, _SC_CHUNK), jnp.int32),
            pltpu.VMEM((nch, _SC_CHUNK, 16), jnp.float32),
            pltpu.SemaphoreType.DMA,
        ],
        compiler_params=pltpu.CompilerParams(use_tc_tiling_on_sc=False),
    )
    def gat(table_hbm, idx_hbm, out_hbm, idx_v, rows_v, sem):
        wid = lax.axis_index("s") * info.num_cores + lax.axis_index("c")
        base = wid * nch
        pltpu.sync_copy(idx_hbm.at[pl.ds(base, nch)], idx_v)
        handles = [
            pltpu.async_copy(table_hbm.at[idx_v.at[j]], rows_v.at[j], sem)
            for j in range(nch)
        ]
        for h in handles:
            h.wait()
        pltpu.sync_copy(rows_v, out_hbm.at[pl.ds(base, nch)])

    return gat(table, gidx2d)


def kernel(xyz):
    B, N, C = xyz.shape
    xyz_only = xyz[:, :, :3]
    xt = jnp.transpose(xyz_only, (2, 0, 1))
    fps_idx, cen = _fps_pallas(xt, _NUM_GROUP)
    cen_b = jnp.transpose(cen, (1, 0, 2))
    xt_b = jnp.transpose(xyz_only, (0, 2, 1))
    center = jnp.transpose(cen, (1, 2, 0))
    c2 = jnp.sum(center ** 2, axis=-1)[:, :, None]
    x2 = jnp.sum(xyz_only ** 2, axis=-1)[:, None, :]
    idx = _knn_pallas(cen_b, xt_b, c2, x2, _GROUP_SIZE, 512)
    table = jnp.pad(xyz_only.reshape(B * N, 3), ((0, 0), (0, 13)))
    b2 = jnp.arange(B, dtype=jnp.int32)[:, None, None]
    gidx = (b2 * N + idx).reshape(-1, _SC_CHUNK)
    rows = _sc_gather(table, gidx)
    neigh_xyz = rows[..., :3].reshape(B, _NUM_GROUP, _GROUP_SIZE, 3)
    neighborhood = neigh_xyz - center[:, :, None, :]
    return (neighborhood, center, idx)


# speedup vs baseline: 1.1312x; 1.0558x over previous
import functools

import jax
import jax.numpy as jnp
from jax import lax
from jax.experimental import pallas as pl
from jax.experimental.pallas import tpu as pltpu
from jax.experimental.pallas import tpu_sc as plsc

_NUM_GROUP = 512
_GROUP_SIZE = 32


def _fps_body(xt_ref, idx_ref, cen_ref, dists_ref, *, n_samples):
    B, N = xt_ref.shape[1], xt_ref.shape[2]
    xs = xt_ref[0]
    ys = xt_ref[1]
    zs = xt_ref[2]
    liota = lax.broadcasted_iota(jnp.int32, (B, N), 1)
    giota = lax.broadcasted_iota(jnp.int32, (B, n_samples), 1)
    dists_ref[...] = jnp.full((B, N), 1e10, dtype=jnp.float32)

    def body(i, carry):
        dists = dists_ref[...]
        m = jnp.max(dists, axis=1, keepdims=True)
        far = jnp.min(jnp.where(dists == m, liota, jnp.int32(N)),
                      axis=1, keepdims=True)
        sel = giota == i
        idx_ref[...] = jnp.where(sel, far, idx_ref[...])
        onehot = liota == far
        zero = jnp.zeros((B, N), jnp.float32)
        cx = jnp.sum(jnp.where(onehot, xs, zero), axis=1, keepdims=True)
        cy = jnp.sum(jnp.where(onehot, ys, zero), axis=1, keepdims=True)
        cz = jnp.sum(jnp.where(onehot, zs, zero), axis=1, keepdims=True)
        cen_ref[0] = jnp.where(sel, cx, cen_ref[0])
        cen_ref[1] = jnp.where(sel, cy, cen_ref[1])
        cen_ref[2] = jnp.where(sel, cz, cen_ref[2])
        dx = xs - cx
        dy = ys - cy
        dz = zs - cz
        d = (dx * dx + dy * dy) + dz * dz
        dists_ref[...] = jnp.minimum(dists, d)
        return carry

    lax.fori_loop(0, n_samples, body, 0)


def _fps_pallas(xt, n_samples):
    _, B, N = xt.shape
    body = functools.partial(_fps_body, n_samples=n_samples)
    idx, cen = pl.pallas_call(
        body,
        out_shape=(
            jax.ShapeDtypeStruct((B, n_samples), jnp.int32),
            jax.ShapeDtypeStruct((3, B, n_samples), jnp.float32),
        ),
        scratch_shapes=[pltpu.VMEM((B, N), jnp.float32)],
    )(xt)
    return idx, cen


def _knn_body(cen_ref, xt_ref, c2_ref, x2_ref, idx_ref, *, k):
    Gt = cen_ref.shape[2]
    N = xt_ref.shape[2]
    c = cen_ref[0]
    xt = xt_ref[0]
    cx = lax.dot_general(c, xt, (((0,), (0,)), ((), ())),
                         preferred_element_type=jnp.float32)
    d = (c2_ref[0] - 2.0 * cx) + x2_ref[0]
    iota = lax.broadcasted_iota(jnp.int32, (Gt, N), 1)
    cols = []
    for _ in range(k):
        amin = jnp.argmin(d, axis=1).astype(jnp.int32)[:, None]
        cols.append(amin)
        d = jnp.where(iota == amin, jnp.float32(jnp.inf), d)
    idx_ref[0] = jnp.concatenate(cols, axis=1)


def _knn_pallas(cen, xt, c2, x2, k, g_tile):
    B, _, G = cen.shape
    N = xt.shape[2]
    body = functools.partial(_knn_body, k=k)
    return pl.pallas_call(
        body,
        grid=(B, G // g_tile),
        in_specs=[
            pl.BlockSpec((1, 3, g_tile), lambda b, g: (b, 0, g)),
            pl.BlockSpec((1, 3, N), lambda b, g: (b, 0, 0)),
            pl.BlockSpec((1, g_tile, 1), lambda b, g: (b, g, 0)),
            pl.BlockSpec((1, 1, N), lambda b, g: (b, 0, 0)),
        ],
        out_specs=pl.BlockSpec((1, g_tile, k), lambda b, g: (b, g, 0)),
        out_shape=jax.ShapeDtypeStruct((B, G, k), jnp.int32),
        compiler_params=pltpu.CompilerParams(
            dimension_semantics=("parallel", "parallel")),
    )(cen, xt, c2, x2)


_SC_CHUNK = 128


def _sc_gather(table, gidx2d):
    nrows = gidx2d.shape[0]
    info = plsc.get_sparse_core_info()
    nw = info.num_cores * info.num_subcores
    nch = nrows // nw

    mesh = plsc.VectorSubcoreMesh(core_axis_name="c", subcore_axis_name="s")

    @functools.partial(
        pl.kernel,
        out_type=jax.ShapeDtypeStruct((nrows, _SC_CHUNK, 16), jnp.float32),
        mesh=mesh,
        scratch_types=[
            pltpu.VMEM((nch, _SC_CHUNK), jnp.int32),
            pltpu.VMEM((nch, _SC_CHUNK, 16), jnp.float32),
            pltpu.SemaphoreType.DMA,
        ],
        compiler_params=pltpu.CompilerParams(use_tc_tiling_on_sc=False),
    )
    def gat(table_hbm, idx_hbm, out_hbm, idx_v, rows_v, sem):
        wid = lax.axis_index("s") * info.num_cores + lax.axis_index("c")
        base = wid * nch
        pltpu.sync_copy(idx_hbm.at[pl.ds(base, nch)], idx_v)
        handles = [
            pltpu.async_copy(table_hbm.at[idx_v.at[j]], rows_v.at[j], sem)
            for j in range(nch)
        ]
        for h in handles:
            h.wait()
        pltpu.sync_copy(rows_v, out_hbm.at[pl.ds(base, nch)])

    return gat(table, gidx2d)


def kernel(xyz):
    B, N, C = xyz.shape
    xyz_only = xyz[:, :, :3]
    xt = jnp.transpose(xyz_only, (2, 0, 1))
    fps_idx, cen = _fps_pallas(xt, _NUM_GROUP)
    cen_b = jnp.transpose(cen, (1, 0, 2))
    xt_b = jnp.transpose(xyz_only, (0, 2, 1))
    center = jnp.transpose(cen, (1, 2, 0))
    c2 = jnp.sum(center ** 2, axis=-1)[:, :, None]
    x2 = jnp.sum(xyz_only ** 2, axis=-1)[:, None, :]
    idx = _knn_pallas(cen_b, xt_b, c2, x2, _GROUP_SIZE, 512)
    table = jnp.pad(xyz_only.reshape(B * N, 3), ((0, 0), (0, 13)))
    b2 = jnp.arange(B, dtype=jnp.int32)[:, None, None]
    gidx = (b2 * N + idx).reshape(-1, _SC_CHUNK)
    rows = _sc_gather(table, gidx)
    neigh_xyz = rows[..., :3].reshape(B, _NUM_GROUP, _GROUP_SIZE, 3)
    neighborhood = neigh_xyz - center[:, :, None, :]
    return (neighborhood, center, idx)
